# Initial kernel scaffold; baseline (speedup 1.0000x reference)
import jax
import jax.numpy as jnp
from jax.experimental import pallas as pl


def kernel(abq_pairs, vals, mask):
    raise NotImplementedError("write your pallas kernel here")


# speedup vs baseline: 1.5538x; 1.5538x over previous
import functools

import jax
import jax.numpy as jnp
from jax.experimental import pallas as pl
from jax.experimental.pallas import tpu as pltpu

_BS, _N, _D, _C = 4, 1024, 3, 128
_M = 256
_NEG = -100.0
_INIT = 100000000.0


def _fps_body(f0_ref, abq_ref, maskf_ref, vals_ref,
              ott_ref, ovals_ref, omask_ref,
              acc_ref, q_ref, qsmem_ref):
    b = pl.program_id(0)

    x0 = abq_ref[0, 0]
    x1 = abq_ref[0, 1]
    x2 = abq_ref[0, 2]
    acc_ref[...] = jnp.sqrt((x0 * x0 + x1 * x1) + x2 * x2)

    maskv = maskf_ref[0]
    lane_iota = jax.lax.broadcasted_iota(jnp.int32, (1, _N), 1)

    def loop(i, carry):
        distances, f = carry
        qsmem_ref[i] = f
        q_ref[pl.ds(i, 1), :] = (lane_iota == f).astype(jnp.float32)
        row = acc_ref[pl.ds(f, 1), :]
        dist = jnp.where(maskv > 0.5, _NEG, row)
        distances = jnp.minimum(distances, dist)
        mx = jnp.max(distances)
        f_new = jnp.min(jnp.where(distances == mx, lane_iota, _N))
        return distances, f_new

    init = (jnp.full((1, _N), _INIT, jnp.float32), f0_ref[b])
    jax.lax.fori_loop(0, _M, loop, init)

    def vals_gather(i, _):
        qi = qsmem_ref[i]
        ovals_ref[0, pl.ds(i, 1), :] = vals_ref[0, pl.ds(qi, 1), :]
        return 0

    jax.lax.fori_loop(0, _M, vals_gather, 0)

    qmat = q_ref[...]
    omask_ref[0] = jax.lax.dot_general(
        maskv, qmat, (((1,), (1,)), ((), ())),
        preferred_element_type=jnp.float32,
        precision=jax.lax.Precision.HIGHEST)

    for d in range(_D):
        def row_gather(i, _, d=d):
            qi = qsmem_ref[i]
            q_ref_row = abq_ref[0, d, pl.ds(qi, 1), :]
            acc_ref[pl.ds(i, 1), :] = q_ref_row
            return 0

        jax.lax.fori_loop(0, _M, row_gather, 0)
        g1 = acc_ref[pl.ds(0, _M), :]
        ott_ref[0, d] = jax.lax.dot_general(
            g1, qmat, (((1,), (1,)), ((), ())),
            preferred_element_type=jnp.float32,
            precision=jax.lax.Precision.HIGHEST)


def _run_fps(abq_t, maskf3, vals, f0):
    grid_spec = pltpu.PrefetchScalarGridSpec(
        num_scalar_prefetch=1,
        grid=(_BS,),
        in_specs=[
            pl.BlockSpec((1, _D, _N, _N), lambda b, f0_ref: (b, 0, 0, 0)),
            pl.BlockSpec((1, 1, _N), lambda b, f0_ref: (b, 0, 0)),
            pl.BlockSpec((1, _N, _C), lambda b, f0_ref: (b, 0, 0)),
        ],
        out_specs=[
            pl.BlockSpec((1, _D, _M, _M), lambda b, f0_ref: (b, 0, 0, 0)),
            pl.BlockSpec((1, _M, _C), lambda b, f0_ref: (b, 0, 0)),
            pl.BlockSpec((1, 1, _M), lambda b, f0_ref: (b, 0, 0)),
        ],
        scratch_shapes=[
            pltpu.VMEM((_N, _N), jnp.float32),
            pltpu.VMEM((_M, _N), jnp.float32),
            pltpu.SMEM((_M,), jnp.int32),
        ],
    )
    return pl.pallas_call(
        _fps_body,
        grid_spec=grid_spec,
        out_shape=[
            jax.ShapeDtypeStruct((_BS, _D, _M, _M), jnp.float32),
            jax.ShapeDtypeStruct((_BS, _M, _C), jnp.float32),
            jax.ShapeDtypeStruct((_BS, 1, _M), jnp.float32),
        ],
    )(f0, abq_t, maskf3, vals)


def kernel(abq_pairs, vals, mask):
    a = jax.random.randint(jax.random.key(42), (_BS,), 0, _N)
    msum = mask.sum(-1)
    k = a.astype(msum.dtype) % msum
    ranks = jnp.cumsum(mask, axis=-1) - 1
    f0 = jnp.argmax(mask & (ranks == k[:, None]), axis=-1).astype(jnp.int32)

    abq_t = jnp.transpose(abq_pairs, (0, 3, 1, 2))
    maskf3 = mask.astype(jnp.float32).reshape(_BS, 1, _N)

    ott, ovals, omaskf = _run_fps(abq_t, maskf3, vals, f0)

    out_abq = jnp.transpose(ott, (0, 3, 2, 1))
    out_mask = omaskf.reshape(_BS, _M) > 0.5
    return (out_abq, ovals, out_mask)


# speedup vs baseline: 2.0373x; 1.3111x over previous
import jax
import jax.numpy as jnp
from jax.experimental import pallas as pl
from jax.experimental.pallas import tpu as pltpu

_BS, _N, _D, _C = 4, 1024, 3, 128
_M = 256
_NEG = -100.0
_INIT = 100000000.0


def _fps_body(f0_ref, abq_ref, maskf_ref, qidx_ref, acc_ref):
    b = pl.program_id(0)
    d = pl.program_id(1)

    x = abq_ref[0, 0]

    @pl.when(d == 0)
    def _():
        acc_ref[b] = x * x

    @pl.when(d != 0)
    def _():
        acc_ref[b] = acc_ref[b] + x * x

    @pl.when((b == _BS - 1) & (d == _D - 1))
    def _():
        acc_ref[...] = jnp.sqrt(acc_ref[...])
        masks = [maskf_ref[bb] for bb in range(_BS)]
        lane_iota = jax.lax.broadcasted_iota(jnp.int32, (1, _N), 1)

        def loop(i, carry):
            dists, fs = carry
            new_d, new_f = [], []
            for bb in range(_BS):
                qidx_ref[bb, i] = fs[bb]
                row = acc_ref[bb, pl.ds(fs[bb], 1), :]
                dist = jnp.where(masks[bb] > 0.5, _NEG, row)
                nd = jnp.minimum(dists[bb], dist)
                mx = jnp.max(nd)
                fn = jnp.min(jnp.where(nd == mx, lane_iota, _N))
                new_d.append(nd)
                new_f.append(fn)
            return tuple(new_d), tuple(new_f)

        init_d = tuple(jnp.full((1, _N), _INIT, jnp.float32)
                       for _ in range(_BS))
        init_f = tuple(f0_ref[bb] for bb in range(_BS))
        jax.lax.fori_loop(0, _M, loop, (init_d, init_f))


def _run_fps(abq_t, maskf3, f0):
    grid_spec = pltpu.PrefetchScalarGridSpec(
        num_scalar_prefetch=1,
        grid=(_BS, _D),
        in_specs=[
            pl.BlockSpec((1, 1, _N, _N), lambda b, d, f0_ref: (b, d, 0, 0)),
            pl.BlockSpec((_BS, 1, _N), lambda b, d, f0_ref: (0, 0, 0)),
        ],
        out_specs=pl.BlockSpec((_BS, _M), lambda b, d, f0_ref: (0, 0),
                               memory_space=pltpu.SMEM),
        scratch_shapes=[
            pltpu.VMEM((_BS, _N, _N), jnp.float32),
        ],
    )
    return pl.pallas_call(
        _fps_body,
        grid_spec=grid_spec,
        out_shape=jax.ShapeDtypeStruct((_BS, _M), jnp.int32),
    )(f0, abq_t, maskf3)


def _gather_body(qsp_ref, abq_ref, qv_ref, maskf_ref, vals_ref,
                 ott_ref, ovals_ref, omask_ref,
                 g1_ref, qt_ref):
    b = pl.program_id(0)
    d = pl.program_id(1)

    @pl.when(d == 0)
    def _():
        row_iota = jax.lax.broadcasted_iota(jnp.int32, (_N, _M), 0)
        qv = qv_ref[0]
        qt_ref[...] = (row_iota == qv).astype(jnp.float32)

        def vals_gather(i, _):
            base = i * 4
            for u in range(4):
                qi = qsp_ref[b, base + u]
                ovals_ref[0, pl.ds(base + u, 1), :] = \
                    vals_ref[0, pl.ds(qi, 1), :]
            return 0

        jax.lax.fori_loop(0, _M // 4, vals_gather, 0)
        omask_ref[0] = jax.lax.dot_general(
            maskf_ref[0], qt_ref[...], (((1,), (0,)), ((), ())),
            preferred_element_type=jnp.float32,
            precision=jax.lax.Precision.HIGHEST)

    def row_gather(i, _):
        base = i * 4
        for u in range(4):
            qi = qsp_ref[b, base + u]
            g1_ref[pl.ds(base + u, 1), :] = abq_ref[0, 0, pl.ds(qi, 1), :]
        return 0

    jax.lax.fori_loop(0, _M // 4, row_gather, 0)
    ott_ref[0, 0] = jax.lax.dot_general(
        g1_ref[...], qt_ref[...], (((1,), (0,)), ((), ())),
        preferred_element_type=jnp.float32,
        precision=jax.lax.Precision.HIGHEST)


def _run_gather(abq_t, qidx, qv3, maskf3, vals):
    grid_spec = pltpu.PrefetchScalarGridSpec(
        num_scalar_prefetch=1,
        grid=(_BS, _D),
        in_specs=[
            pl.BlockSpec((1, 1, _N, _N), lambda b, d, q_ref: (b, d, 0, 0)),
            pl.BlockSpec((1, 1, _M), lambda b, d, q_ref: (b, 0, 0)),
            pl.BlockSpec((1, 1, _N), lambda b, d, q_ref: (b, 0, 0)),
            pl.BlockSpec((1, _N, _C), lambda b, d, q_ref: (b, 0, 0)),
        ],
        out_specs=[
            pl.BlockSpec((1, 1, _M, _M), lambda b, d, q_ref: (b, d, 0, 0)),
            pl.BlockSpec((1, _M, _C), lambda b, d, q_ref: (b, 0, 0)),
            pl.BlockSpec((1, 1, _M), lambda b, d, q_ref: (b, 0, 0)),
        ],
        scratch_shapes=[
            pltpu.VMEM((_M, _N), jnp.float32),
            pltpu.VMEM((_N, _M), jnp.float32),
        ],
    )
    return pl.pallas_call(
        _gather_body,
        grid_spec=grid_spec,
        out_shape=[
            jax.ShapeDtypeStruct((_BS, _D, _M, _M), jnp.float32),
            jax.ShapeDtypeStruct((_BS, _M, _C), jnp.float32),
            jax.ShapeDtypeStruct((_BS, 1, _M), jnp.float32),
        ],
    )(qidx, abq_t, qv3, maskf3, vals)


def kernel(abq_pairs, vals, mask):
    a = jax.random.randint(jax.random.key(42), (_BS,), 0, _N)
    msum = mask.sum(-1)
    k = a.astype(msum.dtype) % msum
    ranks = jnp.cumsum(mask, axis=-1) - 1
    f0 = jnp.argmax(mask & (ranks == k[:, None]), axis=-1).astype(jnp.int32)

    abq_t = jnp.transpose(abq_pairs, (0, 3, 1, 2))
    maskf3 = mask.astype(jnp.float32).reshape(_BS, 1, _N)

    qidx = _run_fps(abq_t, maskf3, f0)
    qv3 = qidx.reshape(_BS, 1, _M)
    ott, ovals, omaskf = _run_gather(abq_t, qidx, qv3, maskf3, vals)

    out_abq = jnp.transpose(ott, (0, 3, 2, 1))
    out_mask = omaskf.reshape(_BS, _M) > 0.5
    return (out_abq, ovals, out_mask)


# speedup vs baseline: 4.7941x; 2.3532x over previous
import jax
import jax.numpy as jnp
from jax.experimental import pallas as pl
from jax.experimental.pallas import tpu as pltpu

_BS, _N, _D, _C = 4, 1024, 3, 128
_M = 256
_NEG = -100.0
_INIT = 100000000.0


def _fps_body(f0_ref, abq_ref, maskf_ref, qidx_ref, acc_ref):
    b = pl.program_id(0)
    d = pl.program_id(1)

    x = abq_ref[0, 0]

    @pl.when(d == 0)
    def _():
        acc_ref[b] = x * x

    @pl.when(d != 0)
    def _():
        acc_ref[b] = acc_ref[b] + x * x

    @pl.when((b == _BS - 1) & (d == _D - 1))
    def _():
        acc_ref[...] = jnp.sqrt(acc_ref[...])
        mbs = [maskf_ref[bb] > 0.5 for bb in range(_BS)]

        def loop(i, carry):
            dists, fs = carry
            new_d, new_f = [], []
            for bb in range(_BS):
                qidx_ref[bb, i] = fs[bb]
                row = acc_ref[bb, pl.ds(fs[bb], 1), :]
                dist = jnp.where(mbs[bb], _NEG, row)
                nd = jnp.minimum(dists[bb], dist)
                fn = jnp.argmax(nd[0]).astype(jnp.int32)
                new_d.append(nd)
                new_f.append(fn)
            return tuple(new_d), tuple(new_f)

        init_d = tuple(jnp.full((1, _N), _INIT, jnp.float32)
                       for _ in range(_BS))
        init_f = tuple(f0_ref[bb] for bb in range(_BS))
        jax.lax.fori_loop(0, _M, loop, (init_d, init_f))


def _run_fps(abq_t, maskf3, f0):
    grid_spec = pltpu.PrefetchScalarGridSpec(
        num_scalar_prefetch=1,
        grid=(_BS, _D),
        in_specs=[
            pl.BlockSpec((1, 1, _N, _N), lambda b, d, f0_ref: (b, d, 0, 0)),
            pl.BlockSpec((_BS, 1, _N), lambda b, d, f0_ref: (0, 0, 0)),
        ],
        out_specs=pl.BlockSpec((_BS, _M), lambda b, d, f0_ref: (0, 0),
                               memory_space=pltpu.SMEM),
        scratch_shapes=[
            pltpu.VMEM((_BS, _N, _N), jnp.float32),
        ],
    )
    return pl.pallas_call(
        _fps_body,
        grid_spec=grid_spec,
        out_shape=jax.ShapeDtypeStruct((_BS, _M), jnp.int32),
    )(f0, abq_t, maskf3)


def _gather_body(qsp_ref, abq_ref, qv_ref, maskf_ref, vals_ref,
                 ott_ref, ovals_ref, omask_ref,
                 g1_ref, qt_ref):
    b = pl.program_id(0)
    d = pl.program_id(1)

    @pl.when(d == 0)
    def _():
        row_iota = jax.lax.broadcasted_iota(jnp.int32, (_N, _M), 0)
        qv = qv_ref[0]
        qt_ref[...] = (row_iota == qv).astype(jnp.float32)

        def vals_gather(i, _):
            base = i * 4
            for u in range(4):
                qi = qsp_ref[b, base + u]
                ovals_ref[0, pl.ds(base + u, 1), :] = \
                    vals_ref[0, pl.ds(qi, 1), :]
            return 0

        jax.lax.fori_loop(0, _M // 4, vals_gather, 0)
        omask_ref[0] = jax.lax.dot_general(
            maskf_ref[0], qt_ref[...], (((1,), (0,)), ((), ())),
            preferred_element_type=jnp.float32,
            precision=jax.lax.Precision.HIGHEST)

    def row_gather(i, _):
        base = i * 4
        for u in range(4):
            qi = qsp_ref[b, base + u]
            g1_ref[pl.ds(base + u, 1), :] = abq_ref[0, 0, pl.ds(qi, 1), :]
        return 0

    jax.lax.fori_loop(0, _M // 4, row_gather, 0)
    ott_ref[0, 0] = jax.lax.dot_general(
        g1_ref[...], qt_ref[...], (((1,), (0,)), ((), ())),
        preferred_element_type=jnp.float32,
        precision=jax.lax.Precision.HIGHEST)


def _run_gather(abq_t, qidx, qv3, maskf3, vals):
    grid_spec = pltpu.PrefetchScalarGridSpec(
        num_scalar_prefetch=1,
        grid=(_BS, _D),
        in_specs=[
            pl.BlockSpec((1, 1, _N, _N), lambda b, d, q_ref: (b, d, 0, 0)),
            pl.BlockSpec((1, 1, _M), lambda b, d, q_ref: (b, 0, 0)),
            pl.BlockSpec((1, 1, _N), lambda b, d, q_ref: (b, 0, 0)),
            pl.BlockSpec((1, _N, _C), lambda b, d, q_ref: (b, 0, 0)),
        ],
        out_specs=[
            pl.BlockSpec((1, 1, _M, _M), lambda b, d, q_ref: (b, d, 0, 0)),
            pl.BlockSpec((1, _M, _C), lambda b, d, q_ref: (b, 0, 0)),
            pl.BlockSpec((1, 1, _M), lambda b, d, q_ref: (b, 0, 0)),
        ],
        scratch_shapes=[
            pltpu.VMEM((_M, _N), jnp.float32),
            pltpu.VMEM((_N, _M), jnp.float32),
        ],
    )
    return pl.pallas_call(
        _gather_body,
        grid_spec=grid_spec,
        out_shape=[
            jax.ShapeDtypeStruct((_BS, _D, _M, _M), jnp.float32),
            jax.ShapeDtypeStruct((_BS, _M, _C), jnp.float32),
            jax.ShapeDtypeStruct((_BS, 1, _M), jnp.float32),
        ],
    )(qidx, abq_t, qv3, maskf3, vals)


def kernel(abq_pairs, vals, mask):
    a = jax.random.randint(jax.random.key(42), (_BS,), 0, _N)
    msum = mask.sum(-1)
    k = a.astype(msum.dtype) % msum
    ranks = jnp.cumsum(mask, axis=-1) - 1
    f0 = jnp.argmax(mask & (ranks == k[:, None]), axis=-1).astype(jnp.int32)

    abq_t = jnp.transpose(abq_pairs, (0, 3, 1, 2))
    maskf3 = mask.astype(jnp.float32).reshape(_BS, 1, _N)

    qidx = _run_fps(abq_t, maskf3, f0)
    qv3 = qidx.reshape(_BS, 1, _M)
    ott, ovals, omaskf = _run_gather(abq_t, qidx, qv3, maskf3, vals)

    out_abq = jnp.transpose(ott, (0, 3, 2, 1))
    out_mask = omaskf.reshape(_BS, _M) > 0.5
    return (out_abq, ovals, out_mask)


# speedup vs baseline: 4.8180x; 1.0050x over previous
import jax
import jax.numpy as jnp
from jax.experimental import pallas as pl
from jax.experimental.pallas import tpu as pltpu

_BS, _N, _D, _C = 4, 1024, 3, 128
_M = 256
_NEG = -100.0
_INIT = 100000000.0


def _fps_body(f0_ref, abq_ref, maskf_ref, qidx_ref, acc_ref):
    b = pl.program_id(0)
    d = pl.program_id(1)

    x = abq_ref[0, 0]

    @pl.when(d == 0)
    def _():
        acc_ref[b] = x * x

    @pl.when(d != 0)
    def _():
        acc_ref[b] = acc_ref[b] + x * x

    @pl.when((b == _BS - 1) & (d == _D - 1))
    def _():
        acc_ref[...] = jnp.sqrt(acc_ref[...])
        mbs = [maskf_ref[bb] > 0.5 for bb in range(_BS)]

        def loop(i, carry):
            dists, fs = carry
            new_d, new_f = [], []
            for bb in range(_BS):
                qidx_ref[bb, i] = fs[bb]
                row = acc_ref[bb, pl.ds(fs[bb], 1), :]
                dist = jnp.where(mbs[bb], _NEG, row)
                nd = jnp.minimum(dists[bb], dist)
                fn = jnp.argmax(nd[0]).astype(jnp.int32)
                new_d.append(nd)
                new_f.append(fn)
            return tuple(new_d), tuple(new_f)

        init_d = tuple(jnp.full((1, _N), _INIT, jnp.float32)
                       for _ in range(_BS))
        init_f = tuple(f0_ref[bb] for bb in range(_BS))
        jax.lax.fori_loop(0, _M, loop, (init_d, init_f), unroll=2)


def _run_fps(abq_t, maskf3, f0):
    grid_spec = pltpu.PrefetchScalarGridSpec(
        num_scalar_prefetch=1,
        grid=(_BS, _D),
        in_specs=[
            pl.BlockSpec((1, 1, _N, _N), lambda b, d, f0_ref: (b, d, 0, 0)),
            pl.BlockSpec((_BS, 1, _N), lambda b, d, f0_ref: (0, 0, 0)),
        ],
        out_specs=pl.BlockSpec((_BS, _M), lambda b, d, f0_ref: (0, 0),
                               memory_space=pltpu.SMEM),
        scratch_shapes=[
            pltpu.VMEM((_BS, _N, _N), jnp.float32),
        ],
    )
    return pl.pallas_call(
        _fps_body,
        grid_spec=grid_spec,
        out_shape=jax.ShapeDtypeStruct((_BS, _M), jnp.int32),
    )(f0, abq_t, maskf3)


def _gather_body(qsp_ref, abq_ref, qv_ref, maskf_ref, vals_ref,
                 ott_ref, ovals_ref, omask_ref,
                 g1_ref, qt_ref):
    b = pl.program_id(0)
    d = pl.program_id(1)

    @pl.when(d == 0)
    def _():
        row_iota = jax.lax.broadcasted_iota(jnp.int32, (_N, _M), 0)
        qv = qv_ref[0]
        qt_ref[...] = (row_iota == qv).astype(jnp.float32)

        def vals_gather(i, _):
            base = i * 4
            for u in range(4):
                qi = qsp_ref[b, base + u]
                ovals_ref[0, pl.ds(base + u, 1), :] = \
                    vals_ref[0, pl.ds(qi, 1), :]
            return 0

        jax.lax.fori_loop(0, _M // 4, vals_gather, 0)
        omask_ref[0] = jax.lax.dot_general(
            maskf_ref[0], qt_ref[...], (((1,), (0,)), ((), ())),
            preferred_element_type=jnp.float32,
            precision=jax.lax.Precision.HIGHEST)

    def row_gather(i, _):
        base = i * 4
        for u in range(4):
            qi = qsp_ref[b, base + u]
            g1_ref[pl.ds(base + u, 1), :] = abq_ref[0, 0, pl.ds(qi, 1), :]
        return 0

    jax.lax.fori_loop(0, _M // 4, row_gather, 0)
    ott_ref[0, 0] = jax.lax.dot_general(
        g1_ref[...], qt_ref[...], (((1,), (0,)), ((), ())),
        preferred_element_type=jnp.float32,
        precision=jax.lax.Precision.HIGHEST)


def _run_gather(abq_t, qidx, qv3, maskf3, vals):
    grid_spec = pltpu.PrefetchScalarGridSpec(
        num_scalar_prefetch=1,
        grid=(_BS, _D),
        in_specs=[
            pl.BlockSpec((1, 1, _N, _N), lambda b, d, q_ref: (b, d, 0, 0)),
            pl.BlockSpec((1, 1, _M), lambda b, d, q_ref: (b, 0, 0)),
            pl.BlockSpec((1, 1, _N), lambda b, d, q_ref: (b, 0, 0)),
            pl.BlockSpec((1, _N, _C), lambda b, d, q_ref: (b, 0, 0)),
        ],
        out_specs=[
            pl.BlockSpec((1, 1, _M, _M), lambda b, d, q_ref: (b, d, 0, 0)),
            pl.BlockSpec((1, _M, _C), lambda b, d, q_ref: (b, 0, 0)),
            pl.BlockSpec((1, 1, _M), lambda b, d, q_ref: (b, 0, 0)),
        ],
        scratch_shapes=[
            pltpu.VMEM((_M, _N), jnp.float32),
            pltpu.VMEM((_N, _M), jnp.float32),
        ],
    )
    return pl.pallas_call(
        _gather_body,
        grid_spec=grid_spec,
        out_shape=[
            jax.ShapeDtypeStruct((_BS, _D, _M, _M), jnp.float32),
            jax.ShapeDtypeStruct((_BS, _M, _C), jnp.float32),
            jax.ShapeDtypeStruct((_BS, 1, _M), jnp.float32),
        ],
    )(qidx, abq_t, qv3, maskf3, vals)


def kernel(abq_pairs, vals, mask):
    a = jax.random.randint(jax.random.key(42), (_BS,), 0, _N)
    msum = mask.sum(-1)
    k = a.astype(msum.dtype) % msum
    ranks = jnp.cumsum(mask, axis=-1) - 1
    f0 = jnp.argmax(mask & (ranks == k[:, None]), axis=-1).astype(jnp.int32)

    abq_t = jnp.transpose(abq_pairs, (0, 3, 1, 2))
    maskf3 = mask.astype(jnp.float32).reshape(_BS, 1, _N)

    qidx = _run_fps(abq_t, maskf3, f0)
    qv3 = qidx.reshape(_BS, 1, _M)
    ott, ovals, omaskf = _run_gather(abq_t, qidx, qv3, maskf3, vals)

    out_abq = jnp.transpose(ott, (0, 3, 2, 1))
    out_mask = omaskf.reshape(_BS, _M) > 0.5
    return (out_abq, ovals, out_mask)
